# baseline (speedup 1.0000x reference)
import jax
import jax.numpy as jnp
from jax.experimental import pallas as pl
from jax.experimental.pallas import tpu as pltpu


def _body(w_ref, sc_ref, out_ref, wt_ref):
    gblk = out_ref.shape[0]
    d = w_ref.shape[1]
    c = w_ref.shape[0]

    @pl.when(pl.program_id(0) == 0)
    def _():
        wt_ref[...] = jnp.transpose(w_ref[...], (1, 0))

    wt = wt_ref[...]
    out_ref[:, :d, :] = jnp.broadcast_to(wt[None, :, :], (gblk, d, c))
    out_ref[:, d:, :] = sc_ref[...][:, None, :]


def kernel(scRNA_count, embedding_weight):
    g, c = scRNA_count.shape
    c2, d = embedding_weight.shape
    assert c2 == c
    gblk = 8

    return pl.pallas_call(
        _body,
        grid=(g // gblk,),
        in_specs=[
            pl.BlockSpec((c, d), lambda i: (0, 0)),
            pl.BlockSpec((gblk, c), lambda i: (i, 0)),
        ],
        out_specs=pl.BlockSpec((gblk, d + 1, c), lambda i: (i, 0, 0)),
        out_shape=jax.ShapeDtypeStruct((g, d + 1, c), jnp.float32),
        scratch_shapes=[pltpu.VMEM((d, c), jnp.float32)],
    )(embedding_weight, scRNA_count)
